# baseline (speedup 1.0000x reference)
import jax
import jax.numpy as jnp
from jax.experimental import pallas as pl

N = 65536
IN_FEATURES = 1024
OUT_FEATURES = 16
BLOCK_N = 2048


def _matmul_body(x_ref, wt_ref, b_ref, out_ref):
    out_ref[...] = (
        jnp.dot(x_ref[...], wt_ref[...], preferred_element_type=jnp.float32)
        + b_ref[...]
    )


def kernel(input, weight, bias):
    wt = weight.T
    b2 = bias.reshape(1, OUT_FEATURES)
    grid = (N // BLOCK_N,)
    return pl.pallas_call(
        _matmul_body,
        grid=grid,
        in_specs=[
            pl.BlockSpec((BLOCK_N, IN_FEATURES), lambda i: (i, 0)),
            pl.BlockSpec((IN_FEATURES, OUT_FEATURES), lambda i: (0, 0)),
            pl.BlockSpec((1, OUT_FEATURES), lambda i: (0, 0)),
        ],
        out_specs=pl.BlockSpec((BLOCK_N, OUT_FEATURES), lambda i: (i, 0)),
        out_shape=jax.ShapeDtypeStruct((N, OUT_FEATURES), jnp.float32),
    )(input, wt, b2)
